# baseline (speedup 1.0000x reference)
import math

import jax
import jax.numpy as jnp
from jax.experimental import pallas as pl

_EPS = 1e-6
_M, _K, _D = 3, 1024, 256
_H, _W = 24, 24
_P = _H * _W
_PB = 192
_HB = _PB // _W
_NPB = _P // _PB


def _vq_body(x_ref, cb_ref, t_ref, g_ref,
             logit_ref, sample_ref, onehot_ref, code_ref):
    xb = x_ref[...]
    cb = cb_ref[0]
    inter = jax.lax.dot_general(
        xb, cb, (((1,), (1,)), ((), ())), preferred_element_type=jnp.float32)
    x2 = jnp.sum(xb * xb, axis=1, keepdims=True)
    c2 = jnp.sum(cb * cb, axis=1)[None, :]
    dist = (x2 + c2) - 2.0 * inter
    tb = jnp.maximum(t_ref[0], _EPS)
    lt = ((-1.0 * dist) / math.sqrt(_K)) * tb
    logit_ref[0] = lt

    iota = jax.lax.broadcasted_iota(jnp.int32, (_PB, _K), 1)

    z = lt + g_ref[0]
    zmax = jnp.max(z, axis=1, keepdims=True)
    zidx = jnp.min(jnp.where(z == zmax, iota, _K), axis=1, keepdims=True)
    sample_ref[0] = (iota == zidx).astype(jnp.float32)

    lmax = jnp.max(lt, axis=1, keepdims=True)
    lidx = jnp.min(jnp.where(lt == lmax, iota, _K), axis=1, keepdims=True)
    onehot_ref[0] = (iota == lidx).astype(jnp.float32)
    code_ref[0] = lidx.reshape(_HB, _W)


_G_CACHE = {}


def _gumbel(n):
    if n not in _G_CACHE:
        with jax.ensure_compile_time_eval():
            _G_CACHE[n] = jax.random.gumbel(
                jax.random.key(42), (n, _M, _H, _W, _K),
                dtype=jnp.float32).reshape(n * _M, _P, _K)
    return _G_CACHE[n]


def kernel(x, codebook, temperature):
    n = x.shape[0]
    nm = n * _M
    xp = jnp.transpose(x, (0, 2, 3, 1)).reshape(n * _P, _M * _D)
    gr = _gumbel(n)
    tr = temperature.reshape(_M, 1, 1)
    _nm_idx = lambda i, j: ((i % n) * _M + i // n, j, 0)
    lt, sample, onehot, code = pl.pallas_call(
        _vq_body,
        grid=(nm, _NPB),
        in_specs=[
            pl.BlockSpec((_PB, _D), lambda i, j: ((i % n) * _NPB + j, i // n)),
            pl.BlockSpec((1, _K, _D), lambda i, j: (i // n, 0, 0)),
            pl.BlockSpec((1, 1, 1), lambda i, j: (i // n, 0, 0)),
            pl.BlockSpec((1, _PB, _K), _nm_idx),
        ],
        out_specs=[
            pl.BlockSpec((1, _PB, _K), _nm_idx),
            pl.BlockSpec((1, _PB, _K), _nm_idx),
            pl.BlockSpec((1, _PB, _K), _nm_idx),
            pl.BlockSpec((1, _HB, _W), _nm_idx),
        ],
        out_shape=[
            jax.ShapeDtypeStruct((nm, _P, _K), jnp.float32),
            jax.ShapeDtypeStruct((nm, _P, _K), jnp.float32),
            jax.ShapeDtypeStruct((nm, _P, _K), jnp.float32),
            jax.ShapeDtypeStruct((nm, _H, _W), jnp.int32),
        ],
    )(xp, codebook, tr, gr)
    shape5 = (n, _M, _H, _W, _K)
    return (sample.reshape(shape5), code.reshape(n, _M, _H, _W),
            onehot.reshape(shape5), lt.reshape(shape5))


# speedup vs baseline: 1.4253x; 1.4253x over previous
import math

import jax
import jax.numpy as jnp
from jax.experimental import pallas as pl

_EPS = 1e-6
_M, _K, _D = 3, 1024, 256
_H, _W = 24, 24
_P = _H * _W


def _vq_body(x_ref, cb_ref, t_ref, g_ref,
             logit_ref, sample_ref, onehot_ref, code_ref):
    xb = x_ref[...]
    cb = cb_ref[0]
    inter = jax.lax.dot_general(
        xb, cb, (((1,), (1,)), ((), ())), preferred_element_type=jnp.float32)
    x2 = jnp.sum(xb * xb, axis=1, keepdims=True)
    c2 = jnp.sum(cb * cb, axis=1)[None, :]
    dist = (x2 + c2) - 2.0 * inter
    tb = jnp.maximum(t_ref[0], _EPS)
    lt = ((-1.0 * dist) / math.sqrt(_K)) * tb
    logit_ref[0] = lt

    iota = jax.lax.broadcasted_iota(jnp.int32, (_P, _K), 1)

    z = lt + g_ref[0]
    zmax = jnp.max(z, axis=1, keepdims=True)
    zidx = jnp.min(jnp.where(z == zmax, iota, _K), axis=1, keepdims=True)
    sample_ref[0] = (iota == zidx).astype(jnp.float32)

    lmax = jnp.max(lt, axis=1, keepdims=True)
    lidx = jnp.min(jnp.where(lt == lmax, iota, _K), axis=1, keepdims=True)
    onehot_ref[0] = (iota == lidx).astype(jnp.float32)
    code_ref[0] = lidx.reshape(_H, _W)


_G_CACHE = {}


def _gumbel(n):
    if n not in _G_CACHE:
        with jax.ensure_compile_time_eval():
            _G_CACHE[n] = jax.random.gumbel(
                jax.random.key(42), (n, _M, _H, _W, _K),
                dtype=jnp.float32).reshape(n * _M, _P, _K)
    return _G_CACHE[n]


def kernel(x, codebook, temperature):
    n = x.shape[0]
    nm = n * _M
    xp = jnp.transpose(x, (0, 2, 3, 1)).reshape(n * _P, _M * _D)
    gr = _gumbel(n)
    tr = temperature.reshape(_M, 1, 1)
    _nm_idx = lambda i: ((i % n) * _M + i // n, 0, 0)
    lt, sample, onehot, code = pl.pallas_call(
        _vq_body,
        grid=(nm,),
        in_specs=[
            pl.BlockSpec((_P, _D), lambda i: (i % n, i // n)),
            pl.BlockSpec((1, _K, _D), lambda i: (i // n, 0, 0)),
            pl.BlockSpec((1, 1, 1), lambda i: (i // n, 0, 0)),
            pl.BlockSpec((1, _P, _K), _nm_idx),
        ],
        out_specs=[
            pl.BlockSpec((1, _P, _K), _nm_idx),
            pl.BlockSpec((1, _P, _K), _nm_idx),
            pl.BlockSpec((1, _P, _K), _nm_idx),
            pl.BlockSpec((1, _H, _W), _nm_idx),
        ],
        out_shape=[
            jax.ShapeDtypeStruct((nm, _P, _K), jnp.float32),
            jax.ShapeDtypeStruct((nm, _P, _K), jnp.float32),
            jax.ShapeDtypeStruct((nm, _P, _K), jnp.float32),
            jax.ShapeDtypeStruct((nm, _H, _W), jnp.int32),
        ],
    )(xp, codebook, tr, gr)
    shape5 = (n, _M, _H, _W, _K)
    return (sample.reshape(shape5), code.reshape(n, _M, _H, _W),
            onehot.reshape(shape5), lt.reshape(shape5))


# speedup vs baseline: 1.4631x; 1.0265x over previous
import math

import jax
import jax.numpy as jnp
from jax.experimental import pallas as pl

_EPS = 1e-6
_M, _K, _D = 3, 1024, 256
_H, _W = 24, 24
_P = _H * _W


def _make_body(n):
    def _vq_body(x_ref, cb_ref, t_ref, g_ref,
                 logit_ref, sample_ref, onehot_ref, code_ref):
        xb = x_ref[...]
        cb = cb_ref[0]
        inter = jax.lax.dot_general(
            xb, cb, (((1,), (1,)), ((), ())),
            preferred_element_type=jnp.float32)
        x2 = jnp.sum(xb * xb, axis=1, keepdims=True)
        c2 = jnp.sum(cb * cb, axis=1)[None, :]
        dist = (x2 + c2) - 2.0 * inter
        m_idx = pl.program_id(0) // n
        lane = jax.lax.broadcasted_iota(jnp.int32, (1, _M), 1)
        tm = jnp.max(jnp.where(lane == m_idx, t_ref[0], -jnp.inf),
                     axis=1, keepdims=True)
        tb = jnp.maximum(tm, _EPS)
        lt = ((-1.0 * dist) / math.sqrt(_K)) * tb
        logit_ref[0] = lt

        iota = jax.lax.broadcasted_iota(jnp.int32, (_P, _K), 1)

        z = lt + g_ref[0]
        zmax = jnp.max(z, axis=1, keepdims=True)
        zidx = jnp.min(jnp.where(z == zmax, iota, _K), axis=1, keepdims=True)
        sample_ref[0] = (iota == zidx).astype(jnp.float32)

        lmax = jnp.max(lt, axis=1, keepdims=True)
        lidx = jnp.min(jnp.where(lt == lmax, iota, _K), axis=1, keepdims=True)
        onehot_ref[0] = (iota == lidx).astype(jnp.float32)
        code_ref[0] = lidx.reshape(_H, _W)

    return _vq_body


_G_CACHE = {}


def _gumbel(n):
    if n not in _G_CACHE:
        with jax.ensure_compile_time_eval():
            _G_CACHE[n] = jax.random.gumbel(
                jax.random.key(42), (n, _M, _H, _W, _K),
                dtype=jnp.float32).reshape(n * _M, _P, _K)
    return _G_CACHE[n]


def kernel(x, codebook, temperature):
    n = x.shape[0]
    nm = n * _M
    xp = jnp.transpose(x, (0, 2, 3, 1)).reshape(n * _P, _M * _D)
    gr = _gumbel(n)
    tr = temperature.reshape(1, 1, _M)
    _nm_idx = lambda i: ((i % n) * _M + i // n, 0, 0)
    lt, sample, onehot, code = pl.pallas_call(
        _make_body(n),
        grid=(nm,),
        in_specs=[
            pl.BlockSpec((_P, _D), lambda i: (i % n, i // n)),
            pl.BlockSpec((1, _K, _D), lambda i: (i // n, 0, 0)),
            pl.BlockSpec((1, 1, _M), lambda i: (0, 0, 0)),
            pl.BlockSpec((1, _P, _K), _nm_idx),
        ],
        out_specs=[
            pl.BlockSpec((1, _P, _K), _nm_idx),
            pl.BlockSpec((1, _P, _K), _nm_idx),
            pl.BlockSpec((1, _P, _K), _nm_idx),
            pl.BlockSpec((1, _H, _W), _nm_idx),
        ],
        out_shape=[
            jax.ShapeDtypeStruct((nm, _P, _K), jnp.float32),
            jax.ShapeDtypeStruct((nm, _P, _K), jnp.float32),
            jax.ShapeDtypeStruct((nm, _P, _K), jnp.float32),
            jax.ShapeDtypeStruct((nm, _H, _W), jnp.int32),
        ],
    )(xp, codebook, tr, gr)
    shape5 = (n, _M, _H, _W, _K)
    return (sample.reshape(shape5), code.reshape(n, _M, _H, _W),
            onehot.reshape(shape5), lt.reshape(shape5))
